# baseline (speedup 1.0000x reference)
import functools

import jax
import jax.numpy as jnp
from jax import lax
from jax.experimental import pallas as pl
from jax.experimental.pallas import tpu as pltpu
from jax.experimental.pallas import tpu_sc as plsc

NC = 2
NS = 16
NW = NC * NS


def _ceil_to(v, m):
    return -(-v // m) * m


def _mm_body(x_ref, w_ref, o_ref):
    o_ref[...] = jnp.dot(x_ref[...], w_ref[...],
                         preferred_element_type=jnp.float32)


def _tc_matmul(x, w, br=512):
    n, ic = x.shape
    oc = w.shape[1]
    return pl.pallas_call(
        _mm_body,
        grid=(pl.cdiv(n, br),),
        in_specs=[pl.BlockSpec((br, ic), lambda i: (i, 0)),
                  pl.BlockSpec((ic, oc), lambda i: (0, 0))],
        out_specs=pl.BlockSpec((br, oc), lambda i: (i, 0)),
        out_shape=jax.ShapeDtypeStruct((n, oc), jnp.float32),
    )(x, w)


def _post_body(*args, nacc, C, oc, enc, softmax):
    acc_parts = [r[0] + r[1] for r in args[:nacc]]
    acc = (jnp.concatenate(acc_parts, axis=1) if nacc > 1
           else acc_parts[0])
    if enc:
        x_ref, r_ref, b_ref, o_ref, c_ref = args[nacc:]
        cnt16 = acc[:, C:C + 16]
        c_ref[...] = cnt16
        cnt = jnp.maximum(cnt16[:, :1], 1.0)
    else:
        cnt_ref, x_ref, r_ref, b_ref, o_ref = args[nacc:]
        cnt = jnp.maximum(cnt_ref[:, :1], 1.0)
    y = acc[:, :C] / cnt
    y = y + jnp.dot(x_ref[...], r_ref[...],
                    preferred_element_type=jnp.float32) + b_ref[...]
    if softmax:
        col = lax.broadcasted_iota(jnp.int32, y.shape, 1)
        y = jnp.where(col < oc, y, -jnp.inf)
        m = jnp.max(y, axis=1, keepdims=True)
        e = jnp.exp(y - m)
        y = e / jnp.sum(e, axis=1, keepdims=True)
    else:
        y = jnp.maximum(y, 0.0)
    o_ref[...] = y


_SPMEM_WORDS = 2_097_151


def _pick_ch(N, E, W, W2):
    R = _ceil_to(N + 1, NS * 8)
    best = 16
    for ch in (128, 64, 32, 16):
        epad = _ceil_to(E, NW * ch * 2)
        epw = epad // NW
        pertile = 4 * epw + 2 * ch * (W + W2) + 3 * (ch + 16) + 1500
        if pertile * 4 > 420_000:
            continue
        if R * W2 + 16 * pertile + 60_000 > _SPMEM_WORDS:
            continue
        return ch
    return best


def _sc_conv(N, E, C, with_count):
    W = 3 * C
    W2 = C + 16 if with_count else C
    CH = _pick_ch(N, E, W, W2)
    EPAD = _ceil_to(E, NW * CH * 2)
    EPW = EPAD // NW
    NCHUNK = EPW // CH
    R = _ceil_to(N + 1, NS * 8)
    RPS = R // NS
    mesh = plsc.VectorSubcoreMesh(core_axis_name="c", subcore_axis_name="s",
                                  num_cores=NC, num_subcores=NS)

    @functools.partial(
        pl.kernel,
        out_type=jax.ShapeDtypeStruct((NC, R, W2), jnp.float32),
        mesh=mesh,
        scratch_types=[
            pltpu.VMEM((NCHUNK, CH), jnp.int32),
            pltpu.VMEM((NCHUNK, CH), jnp.int32),
            pltpu.VMEM((EPW,), jnp.float32),
            pltpu.VMEM((EPW,), jnp.float32),
            pltpu.VMEM((CH, W), jnp.float32),
            pltpu.VMEM((CH, W), jnp.float32),
            pltpu.VMEM((CH, W2), jnp.float32),
            pltpu.VMEM((CH, W2), jnp.float32),
            pltpu.VMEM((3, CH + 16), jnp.float32),
            pltpu.VMEM((12, 16), jnp.float32),
            pltpu.VMEM_SHARED((R, W2), jnp.float32),
            pltpu.SemaphoreType.DMA,
            pltpu.SemaphoreType.DMA,
        ],
        compiler_params=pltpu.CompilerParams(use_tc_tiling_on_sc=False,
                                            needs_layout_passes=False),
    )
    def k(src2_h, dst2_h, p0_h, p1_h, coef_h, xg_h, out_h,
          src_a, dst_a, p0_a, p1_a, rows0, rows1, msg0, msg1, gbuf,
          coef_v, acc_sh, sem0, sem1):
        cid = lax.axis_index("c")
        sid = lax.axis_index("s")
        wid = sid * NC + cid
        rows = (rows0, rows1)
        msgs = (msg0, msg1)
        sems = (sem0, sem1)
        zero16 = jnp.zeros((16,), jnp.float32)

        def zrow(e, carry):
            for cb in range(W2 // 16):
                msg0[e, pl.ds(cb * 16, 16)] = zero16
            return carry

        lax.fori_loop(0, CH, zrow, 0)

        r0 = sid * RPS
        nfull, rem = divmod(RPS, CH)
        for j in range(nfull):
            pltpu.sync_copy(msg0, acc_sh.at[pl.ds(r0 + j * CH, CH)])
        if rem:
            pltpu.sync_copy(msg0.at[pl.ds(0, rem)],
                            acc_sh.at[pl.ds(r0 + nfull * CH, rem)])
        if with_count:
            one16 = jnp.ones((16,), jnp.float32)

            def orow(e, carry):
                msg0[e, pl.ds(C, 16)] = one16
                msg1[e, pl.ds(C, 16)] = one16
                return carry

            lax.fori_loop(0, CH, orow, 0)
        pltpu.sync_copy(coef_h, coef_v)
        pltpu.sync_copy(src2_h.at[pl.ds(wid * NCHUNK, NCHUNK)], src_a)
        pltpu.sync_copy(dst2_h.at[pl.ds(wid * NCHUNK, NCHUNK)], dst_a)
        pltpu.sync_copy(p0_h.at[pl.ds(wid * EPW, EPW)], p0_a)
        pltpu.sync_copy(p1_h.at[pl.ds(wid * EPW, EPW)], p1_a)
        plsc.subcore_barrier()

        pltpu.async_copy(xg_h.at[src_a.at[0]], rows0, sem0)
        pltpu.async_copy(xg_h.at[src_a.at[1]], rows1, sem1)

        def pair(i, carry):
            for b in (0, 1):
                ci = 2 * i + b
                pltpu.make_async_copy(xg_h.at[src_a.at[b]], rows[b],
                                      sems[b]).wait()
                off = ci * CH
                rv = rows[b]
                mv = msgs[b]
                iota16 = lax.iota(jnp.int32, 16)

                def _gauss3(j):
                    p0 = p0_a[pl.ds(off + j * 16, 16)]
                    p1 = p1_a[pl.ds(off + j * 16, 16)]
                    g = []
                    for kk in range(3):
                        d0 = p0 - coef_v[kk]
                        d1 = p1 - coef_v[3 + kk]
                        g.append(jnp.exp(coef_v[6 + kk] * d0 * d0
                                         + coef_v[9 + kk] * d1 * d1))
                    return g

                if False:
                    def group(j, c2):
                        g = _gauss3(j)
                        row_ids = j * 16 + iota16
                        for c in range(C):
                            v = g[0] * plsc.load_gather(
                                rv, [row_ids,
                                     jnp.full((16,), c, jnp.int32)])
                            v += g[1] * plsc.load_gather(
                                rv, [row_ids,
                                     jnp.full((16,), C + c, jnp.int32)])
                            v += g[2] * plsc.load_gather(
                                rv, [row_ids,
                                     jnp.full((16,), 2 * C + c, jnp.int32)])
                            plsc.store_scatter(
                                mv, [row_ids,
                                     jnp.full((16,), c, jnp.int32)], v)
                        return c2

                    lax.fori_loop(0, CH // 16, group, 0)
                else:
                    for j in range(CH // 16):
                        g = _gauss3(j)
                        for kk in range(3):
                            gbuf[kk, pl.ds(j * 16, 16)] = g[kk]

                    def erow(e, c2):
                        g0 = gbuf[0, pl.ds(e, 16)][0]
                        g1 = gbuf[1, pl.ds(e, 16)][0]
                        g2 = gbuf[2, pl.ds(e, 16)][0]
                        for cb in range(C // 16):
                            v = (g0 * rv[e, pl.ds(cb * 16, 16)]
                                 + g1 * rv[e, pl.ds(C + cb * 16, 16)]
                                 + g2 * rv[e, pl.ds(2 * C + cb * 16, 16)])
                            mv[e, pl.ds(cb * 16, 16)] = v
                        return c2

                    lax.fori_loop(0, CH, erow, 0,
                                  unroll=(8 if C <= 32 else
                                          4 if C <= 128 else 1))
                pltpu.sync_copy(mv, acc_sh.at[dst_a.at[ci]], add=True)
                if NCHUNK > 2:
                    @pl.when(ci + 2 < NCHUNK)
                    def _():
                        pltpu.async_copy(xg_h.at[src_a.at[ci + 2]],
                                         rows[b], sems[b])
            return carry

        lax.fori_loop(0, NCHUNK // 2, pair, 0)
        plsc.subcore_barrier()
        pltpu.sync_copy(acc_sh.at[pl.ds(r0, RPS)],
                        out_h.at[cid, pl.ds(r0, RPS)])

    return k, EPAD, CH


def _sc_hexup(n_new, C, CH):
    NPAD = _ceil_to(n_new, NW * CH * 2)
    NPW = NPAD // NW
    NCHUNK = NPW // CH
    mesh = plsc.VectorSubcoreMesh(core_axis_name="c", subcore_axis_name="s",
                                  num_cores=NC, num_subcores=NS)

    @functools.partial(
        pl.kernel,
        out_type=jax.ShapeDtypeStruct((NPAD, C), jnp.float32),
        mesh=mesh,
        scratch_types=[
            pltpu.VMEM((NCHUNK, CH), jnp.int32),
            pltpu.VMEM((NCHUNK, CH), jnp.int32),
            pltpu.VMEM((CH, C), jnp.float32),
            pltpu.VMEM((CH, C), jnp.float32),
            pltpu.VMEM((CH, C), jnp.float32),
            pltpu.VMEM((CH, C), jnp.float32),
            pltpu.SemaphoreType.DMA,
            pltpu.SemaphoreType.DMA,
        ],
        compiler_params=pltpu.CompilerParams(use_tc_tiling_on_sc=False,
                                            needs_layout_passes=False),
    )
    def k(u02_h, u12_h, h_h, out_h, u0_a, u1_a,
          ra0, ra1, rb0, rb1, sem0, sem1):
        cid = lax.axis_index("c")
        sid = lax.axis_index("s")
        wid = sid * NC + cid
        base_w = wid * NPW
        ras = (ra0, ra1)
        rbs = (rb0, rb1)
        sems = (sem0, sem1)
        pltpu.sync_copy(u02_h.at[pl.ds(wid * NCHUNK, NCHUNK)], u0_a)
        pltpu.sync_copy(u12_h.at[pl.ds(wid * NCHUNK, NCHUNK)], u1_a)
        pltpu.async_copy(h_h.at[u0_a.at[0]], ra0, sem0)
        pltpu.async_copy(h_h.at[u1_a.at[0]], rb0, sem0)
        pltpu.async_copy(h_h.at[u0_a.at[1]], ra1, sem1)
        pltpu.async_copy(h_h.at[u1_a.at[1]], rb1, sem1)

        def pair(i, carry):
            for b in (0, 1):
                ci = 2 * i + b
                pltpu.make_async_copy(h_h.at[u0_a.at[b]], ras[b],
                                      sems[b]).wait()
                pltpu.make_async_copy(h_h.at[u1_a.at[b]], rbs[b],
                                      sems[b]).wait()
                ra = ras[b]
                rb = rbs[b]

                def erow(e, c2):
                    for cb in range(C // 16):
                        s = pl.ds(cb * 16, 16)
                        ra[e, s] = (ra[e, s] + rb[e, s]) * 0.5
                    return c2

                lax.fori_loop(0, CH, erow, 0,
                              unroll=(4 if C <= 128 else 1))
                pltpu.sync_copy(ra, out_h.at[pl.ds(base_w + ci * CH, CH)])
                if NCHUNK > 2:
                    @pl.when(ci + 2 < NCHUNK)
                    def _():
                        pltpu.async_copy(h_h.at[u0_a.at[ci + 2]],
                                         ras[b], sems[b])
                        pltpu.async_copy(h_h.at[u1_a.at[ci + 2]],
                                         rbs[b], sems[b])
            return carry

        lax.fori_loop(0, NCHUNK // 2, pair, 0)

    return k, NPAD


def _gmm_prep(p, oc):
    ic = p["g"].shape[0]
    C = _ceil_to(oc, 16)
    g3 = p["g"].reshape(ic, 3, oc)
    g_pad = jnp.zeros((ic, 3, C), jnp.float32).at[:, :, :oc].set(g3)
    root_pad = jnp.zeros((ic, C), jnp.float32).at[:, :oc].set(p["root"])
    bias_pad = jnp.zeros((1, C), jnp.float32).at[0, :oc].set(p["bias"])
    a = -0.5 / (1e-14 + p["sigma"] ** 2)
    coef = jnp.concatenate([p["mu"][:, 0], p["mu"][:, 1], a[:, 0], a[:, 1]])
    coef = jnp.broadcast_to(coef[:, None], (12, 16)).astype(jnp.float32)
    return C, g_pad, root_pad, bias_pad, coef


_ACC_BUDGET_WORDS = 1_350_000


def _gmm(x_in, e, pseudo, p, oc, cnt16=None, softmax=False):
    N, ic = x_in.shape
    E = e.shape[1]
    C, g3_pad, root_pad, bias_pad, coef = _gmm_prep(p, oc)
    with_count = cnt16 is None
    R = _ceil_to(N + 1, NS * 8)
    maxW2 = max(16, (_ACC_BUDGET_WORDS // R) // 16 * 16)
    parts = []
    rem = C
    while rem > 0:
        take = min(rem, maxW2)
        rem -= take
        parts.append(take)
    if with_count and parts[-1] + 16 > maxW2:
        parts[-1] -= 16
        parts.append(16)
        if parts[0] == 0:
            parts = parts[1:]
    accs = []
    off = 0
    cache = {}
    for pi, Cp in enumerate(parts):
        cnt_p = with_count and pi == len(parts) - 1
        kfn, EPAD, CH = _sc_conv(N, E, Cp, cnt_p)
        key = (EPAD, CH)
        if key not in cache:
            pad = EPAD - E
            cache[key] = (
                jnp.pad(e[0], (0, pad)).reshape(EPAD // CH, CH),
                jnp.pad(e[1], (0, pad), constant_values=N
                        ).reshape(EPAD // CH, CH),
                jnp.pad(pseudo[:, 0], (0, pad)),
                jnp.pad(pseudo[:, 1], (0, pad)))
        src2, dst2, p0, p1 = cache[key]
        g_part = g3_pad[:, :, off:off + Cp].reshape(ic, 3 * Cp)
        xg = _tc_matmul(x_in, g_part)
        accs.append(kfn(src2, dst2, p0, p1, coef, xg))
        off += Cp
    nacc = len(accs)
    BR = 256
    acc_specs = [pl.BlockSpec((NC, BR, a.shape[2]), lambda i: (0, i, 0))
                 for a in accs]
    if with_count:
        out, cnt_out = pl.pallas_call(
            functools.partial(_post_body, nacc=nacc, C=C, oc=oc,
                              enc=True, softmax=softmax),
            grid=(pl.cdiv(N, BR),),
            in_specs=acc_specs + [
                pl.BlockSpec((BR, ic), lambda i: (i, 0)),
                pl.BlockSpec((ic, C), lambda i: (0, 0)),
                pl.BlockSpec((1, C), lambda i: (0, 0))],
            out_specs=[pl.BlockSpec((BR, C), lambda i: (i, 0)),
                       pl.BlockSpec((BR, 16), lambda i: (i, 0))],
            out_shape=[jax.ShapeDtypeStruct((N, C), jnp.float32),
                       jax.ShapeDtypeStruct((N, 16), jnp.float32)],
        )(*accs, x_in, root_pad, bias_pad)
        return out[:, :oc], cnt_out
    out = pl.pallas_call(
        functools.partial(_post_body, nacc=nacc, C=C, oc=oc,
                          enc=False, softmax=softmax),
        grid=(pl.cdiv(N, BR),),
        in_specs=acc_specs + [
            pl.BlockSpec((BR, 16), lambda i: (i, 0)),
            pl.BlockSpec((BR, ic), lambda i: (i, 0)),
            pl.BlockSpec((ic, C), lambda i: (0, 0)),
            pl.BlockSpec((1, C), lambda i: (0, 0))],
        out_specs=pl.BlockSpec((BR, C), lambda i: (i, 0)),
        out_shape=jax.ShapeDtypeStruct((N, C), jnp.float32),
    )(*accs, cnt16, x_in, root_pad, bias_pad)
    return out[:, :oc], None


def _hexup(h, up):
    n_new = up.shape[0]
    C = h.shape[1]
    if n_new % (NW * 64) == 0:
        CH = 64
    elif n_new % (NW * 48) == 0:
        CH = 48
    else:
        CH = 16
    kfn, NPAD = _sc_hexup(n_new, C, CH)
    pad = NPAD - n_new
    u0 = jnp.pad(up[:, 0], (0, pad)).reshape(NPAD // CH, CH)
    u1 = jnp.pad(up[:, 1], (0, pad)).reshape(NPAD // CH, CH)
    out = kfn(u0, u1, h)
    return jnp.concatenate([h, out[:n_new]], axis=0)


def _pool(x):
    return x[:(x.shape[0] + 6) // 4]


def kernel(x, e1, e2, e3, e4, e5, e6,
           pseudo1, pseudo2, pseudo3, pseudo4, pseudo5, pseudo6,
           up2, up3, up4, up5, up6, params):
    x0 = x
    h, cnt6 = _gmm(x0, e6, pseudo6, params["conv1"], 32)
    x1 = _pool(h)
    h, cnt5 = _gmm(x1, e5, pseudo5, params["conv2"], 64)
    x2 = _pool(h)
    h, cnt4 = _gmm(x2, e4, pseudo4, params["conv3"], 128)
    x3 = _pool(h)
    h, cnt3 = _gmm(x3, e3, pseudo3, params["conv4"], 256)
    x4 = _pool(h)
    h, cnt2 = _gmm(x4, e2, pseudo2, params["conv5"], 512)
    x5 = _pool(h)
    h, _ = _gmm(x5, e1, pseudo1, params["conv6"], 512)
    h = _hexup(h, up2)
    h = jnp.concatenate([h, x4], axis=1)
    h, _ = _gmm(h, e2, pseudo2, params["conv7"], 256, cnt16=cnt2)
    h = _hexup(h, up3)
    h = jnp.concatenate([h, x3], axis=1)
    h, _ = _gmm(h, e3, pseudo3, params["conv8"], 128, cnt16=cnt3)
    h = _hexup(h, up4)
    h = jnp.concatenate([h, x2], axis=1)
    h, _ = _gmm(h, e4, pseudo4, params["conv9"], 64, cnt16=cnt4)
    h = _hexup(h, up5)
    h = jnp.concatenate([h, x1], axis=1)
    h, _ = _gmm(h, e5, pseudo5, params["conv10"], 32, cnt16=cnt5)
    h = _hexup(h, up6)
    h = jnp.concatenate([h, x0], axis=1)
    h, _ = _gmm(h, e6, pseudo6, params["conv11"], 37, cnt16=cnt6,
                softmax=True)
    return h


# speedup vs baseline: 1.0581x; 1.0581x over previous
import functools

import jax
import jax.numpy as jnp
from jax import lax
from jax.experimental import pallas as pl
from jax.experimental.pallas import tpu as pltpu
from jax.experimental.pallas import tpu_sc as plsc

NC = 2
NS = 16
NW = NC * NS


def _ceil_to(v, m):
    return -(-v // m) * m


def _mm_body(x_ref, w_ref, o_ref):
    o_ref[...] = jnp.dot(x_ref[...], w_ref[...],
                         preferred_element_type=jnp.float32)


def _tc_matmul(x, w, br=512):
    n, ic = x.shape
    oc = w.shape[1]
    return pl.pallas_call(
        _mm_body,
        grid=(pl.cdiv(n, br),),
        in_specs=[pl.BlockSpec((br, ic), lambda i: (i, 0)),
                  pl.BlockSpec((ic, oc), lambda i: (0, 0))],
        out_specs=pl.BlockSpec((br, oc), lambda i: (i, 0)),
        out_shape=jax.ShapeDtypeStruct((n, oc), jnp.float32),
    )(x, w)


def _post_body(*args, nacc, C, oc, enc, softmax):
    acc_parts = [r[0] + r[1] for r in args[:nacc]]
    acc = (jnp.concatenate(acc_parts, axis=1) if nacc > 1
           else acc_parts[0])
    if enc:
        x_ref, r_ref, b_ref, o_ref, c_ref = args[nacc:]
        cnt16 = acc[:, C:C + 16]
        c_ref[...] = cnt16
        cnt = jnp.maximum(cnt16[:, :1], 1.0)
    else:
        cnt_ref, x_ref, r_ref, b_ref, o_ref = args[nacc:]
        cnt = jnp.maximum(cnt_ref[:, :1], 1.0)
    y = acc[:, :C] / cnt
    y = y + jnp.dot(x_ref[...], r_ref[...],
                    preferred_element_type=jnp.float32) + b_ref[...]
    if softmax:
        col = lax.broadcasted_iota(jnp.int32, y.shape, 1)
        y = jnp.where(col < oc, y, -jnp.inf)
        m = jnp.max(y, axis=1, keepdims=True)
        e = jnp.exp(y - m)
        y = e / jnp.sum(e, axis=1, keepdims=True)
    else:
        y = jnp.maximum(y, 0.0)
    o_ref[...] = y


_SPMEM_WORDS = 2_097_151


def _pick_ch(N, E, W, W2):
    R = _ceil_to(N + 1, NS * 8)
    best = 16
    for ch in (128, 64, 32, 16):
        epad = _ceil_to(E, NW * ch * 2)
        epw = epad // NW
        pertile = 4 * epw + 2 * ch * (W + W2) + 3 * (ch + 16) + 1500
        if pertile * 4 > 420_000:
            continue
        if R * W2 + 16 * pertile + 60_000 > _SPMEM_WORDS:
            continue
        return ch
    return best


def _sc_conv(N, E, C, with_count):
    W = 3 * C
    W2 = C + 16 if with_count else C
    CH = _pick_ch(N, E, W, W2)
    EPAD = _ceil_to(E, NW * CH * 2)
    EPW = EPAD // NW
    NCHUNK = EPW // CH
    R = _ceil_to(N + 1, NS * 8)
    RPS = R // NS
    mesh = plsc.VectorSubcoreMesh(core_axis_name="c", subcore_axis_name="s",
                                  num_cores=NC, num_subcores=NS)

    @functools.partial(
        pl.kernel,
        out_type=jax.ShapeDtypeStruct((NC, R, W2), jnp.float32),
        mesh=mesh,
        scratch_types=[
            pltpu.VMEM((NCHUNK, CH), jnp.int32),
            pltpu.VMEM((NCHUNK, CH), jnp.int32),
            pltpu.VMEM((EPW,), jnp.float32),
            pltpu.VMEM((EPW,), jnp.float32),
            pltpu.VMEM((CH, W), jnp.float32),
            pltpu.VMEM((CH, W), jnp.float32),
            pltpu.VMEM((CH, W2), jnp.float32),
            pltpu.VMEM((CH, W2), jnp.float32),
            pltpu.VMEM((3, CH + 16), jnp.float32),
            pltpu.VMEM((12, 16), jnp.float32),
            pltpu.VMEM_SHARED((R, W2), jnp.float32),
            pltpu.SemaphoreType.DMA,
            pltpu.SemaphoreType.DMA,
        ],
        compiler_params=pltpu.CompilerParams(use_tc_tiling_on_sc=False,
                                            needs_layout_passes=False),
    )
    def k(src2_h, dst2_h, p0_h, p1_h, coef_h, xg_h, out_h,
          src_a, dst_a, p0_a, p1_a, rows0, rows1, msg0, msg1, gbuf,
          coef_v, acc_sh, sem0, sem1):
        cid = lax.axis_index("c")
        sid = lax.axis_index("s")
        wid = sid * NC + cid
        rows = (rows0, rows1)
        msgs = (msg0, msg1)
        sems = (sem0, sem1)
        zero16 = jnp.zeros((16,), jnp.float32)

        def zrow(e, carry):
            for cb in range(W2 // 16):
                msg0[e, pl.ds(cb * 16, 16)] = zero16
            return carry

        lax.fori_loop(0, CH, zrow, 0)

        r0 = sid * RPS
        nfull, rem = divmod(RPS, CH)
        for j in range(nfull):
            pltpu.sync_copy(msg0, acc_sh.at[pl.ds(r0 + j * CH, CH)])
        if rem:
            pltpu.sync_copy(msg0.at[pl.ds(0, rem)],
                            acc_sh.at[pl.ds(r0 + nfull * CH, rem)])
        if with_count:
            one16 = jnp.ones((16,), jnp.float32)

            def orow(e, carry):
                msg0[e, pl.ds(C, 16)] = one16
                msg1[e, pl.ds(C, 16)] = one16
                return carry

            lax.fori_loop(0, CH, orow, 0)
        pltpu.sync_copy(coef_h, coef_v)
        pltpu.sync_copy(src2_h.at[pl.ds(wid * NCHUNK, NCHUNK)], src_a)
        pltpu.sync_copy(dst2_h.at[pl.ds(wid * NCHUNK, NCHUNK)], dst_a)
        pltpu.sync_copy(p0_h.at[pl.ds(wid * EPW, EPW)], p0_a)
        pltpu.sync_copy(p1_h.at[pl.ds(wid * EPW, EPW)], p1_a)
        plsc.subcore_barrier()

        pltpu.async_copy(xg_h.at[src_a.at[0]], rows0, sem0)
        pltpu.async_copy(xg_h.at[src_a.at[1]], rows1, sem1)

        def pair(i, carry):
            for b in (0, 1):
                ci = 2 * i + b
                pltpu.make_async_copy(xg_h.at[src_a.at[b]], rows[b],
                                      sems[b]).wait()
                off = ci * CH
                rv = rows[b]
                mv = msgs[b]
                iota16 = lax.iota(jnp.int32, 16)

                def _gauss3(j):
                    p0 = p0_a[pl.ds(off + j * 16, 16)]
                    p1 = p1_a[pl.ds(off + j * 16, 16)]
                    g = []
                    for kk in range(3):
                        d0 = p0 - coef_v[kk]
                        d1 = p1 - coef_v[3 + kk]
                        g.append(jnp.exp(coef_v[6 + kk] * d0 * d0
                                         + coef_v[9 + kk] * d1 * d1))
                    return g

                if False:
                    def group(j, c2):
                        g = _gauss3(j)
                        row_ids = j * 16 + iota16
                        for c in range(C):
                            v = g[0] * plsc.load_gather(
                                rv, [row_ids,
                                     jnp.full((16,), c, jnp.int32)])
                            v += g[1] * plsc.load_gather(
                                rv, [row_ids,
                                     jnp.full((16,), C + c, jnp.int32)])
                            v += g[2] * plsc.load_gather(
                                rv, [row_ids,
                                     jnp.full((16,), 2 * C + c, jnp.int32)])
                            plsc.store_scatter(
                                mv, [row_ids,
                                     jnp.full((16,), c, jnp.int32)], v)
                        return c2

                    lax.fori_loop(0, CH // 16, group, 0)
                else:
                    for j in range(CH // 16):
                        g = _gauss3(j)
                        for kk in range(3):
                            gbuf[kk, pl.ds(j * 16, 16)] = g[kk]

                    def erow(e, c2):
                        g0 = gbuf[0, pl.ds(e, 16)][0]
                        g1 = gbuf[1, pl.ds(e, 16)][0]
                        g2 = gbuf[2, pl.ds(e, 16)][0]
                        for cb in range(C // 16):
                            v = (g0 * rv[e, pl.ds(cb * 16, 16)]
                                 + g1 * rv[e, pl.ds(C + cb * 16, 16)]
                                 + g2 * rv[e, pl.ds(2 * C + cb * 16, 16)])
                            mv[e, pl.ds(cb * 16, 16)] = v
                        return c2

                    lax.fori_loop(0, CH, erow, 0,
                                  unroll=(8 if C <= 32 else
                                          4 if C <= 128 else 1))
                pltpu.sync_copy(mv, acc_sh.at[dst_a.at[ci]], add=True)
                if NCHUNK > 2:
                    @pl.when(ci + 2 < NCHUNK)
                    def _():
                        pltpu.async_copy(xg_h.at[src_a.at[ci + 2]],
                                         rows[b], sems[b])
            return carry

        lax.fori_loop(0, NCHUNK // 2, pair, 0)
        plsc.subcore_barrier()
        pltpu.sync_copy(acc_sh.at[pl.ds(r0, RPS)],
                        out_h.at[cid, pl.ds(r0, RPS)])

    return k, EPAD, CH


def _sc_hexup(n_new, C, CH):
    NPAD = _ceil_to(n_new, NW * CH)
    NPW = NPAD // NW
    NCHUNK = NPW // CH
    mesh = plsc.VectorSubcoreMesh(core_axis_name="c", subcore_axis_name="s",
                                  num_cores=NC, num_subcores=NS)

    @functools.partial(
        pl.kernel,
        out_type=jax.ShapeDtypeStruct((NPAD, C), jnp.float32),
        mesh=mesh,
        scratch_types=[
            pltpu.VMEM((NCHUNK, CH), jnp.int32),
            pltpu.VMEM((NCHUNK, CH), jnp.int32),
            pltpu.VMEM((CH, C), jnp.float32),
            pltpu.VMEM((CH, C), jnp.float32),
            pltpu.SemaphoreType.DMA,
        ],
        compiler_params=pltpu.CompilerParams(use_tc_tiling_on_sc=False,
                                            needs_layout_passes=False),
    )
    def k(u02_h, u12_h, h_h, out_h, u0_a, u1_a, r0_v, r1_v, sem):
        cid = lax.axis_index("c")
        sid = lax.axis_index("s")
        wid = sid * NC + cid
        base_w = wid * NPW
        pltpu.sync_copy(u02_h.at[pl.ds(wid * NCHUNK, NCHUNK)], u0_a)
        pltpu.sync_copy(u12_h.at[pl.ds(wid * NCHUNK, NCHUNK)], u1_a)

        def chunk(ci, carry):
            pltpu.async_copy(h_h.at[u0_a.at[ci]], r0_v, sem)
            pltpu.async_copy(h_h.at[u1_a.at[ci]], r1_v, sem)
            pltpu.make_async_copy(h_h.at[u0_a.at[ci]], r0_v, sem).wait()
            pltpu.make_async_copy(h_h.at[u1_a.at[ci]], r1_v, sem).wait()

            def erow(e, c2):
                for cb in range(C // 16):
                    s = pl.ds(cb * 16, 16)
                    r0_v[e, s] = (r0_v[e, s] + r1_v[e, s]) * 0.5
                return c2

            lax.fori_loop(0, CH, erow, 0)
            pltpu.sync_copy(r0_v, out_h.at[pl.ds(base_w + ci * CH, CH)])
            return carry

        lax.fori_loop(0, NCHUNK, chunk, 0)

    return k, NPAD


def _gmm_prep(p, oc):
    ic = p["g"].shape[0]
    C = _ceil_to(oc, 16)
    g3 = p["g"].reshape(ic, 3, oc)
    g_pad = jnp.zeros((ic, 3, C), jnp.float32).at[:, :, :oc].set(g3)
    root_pad = jnp.zeros((ic, C), jnp.float32).at[:, :oc].set(p["root"])
    bias_pad = jnp.zeros((1, C), jnp.float32).at[0, :oc].set(p["bias"])
    a = -0.5 / (1e-14 + p["sigma"] ** 2)
    coef = jnp.concatenate([p["mu"][:, 0], p["mu"][:, 1], a[:, 0], a[:, 1]])
    coef = jnp.broadcast_to(coef[:, None], (12, 16)).astype(jnp.float32)
    return C, g_pad, root_pad, bias_pad, coef


_ACC_BUDGET_WORDS = 1_350_000


def _gmm(x_in, e, pseudo, p, oc, cnt16=None, softmax=False):
    N, ic = x_in.shape
    E = e.shape[1]
    C, g3_pad, root_pad, bias_pad, coef = _gmm_prep(p, oc)
    with_count = cnt16 is None
    R = _ceil_to(N + 1, NS * 8)
    maxW2 = max(16, (_ACC_BUDGET_WORDS // R) // 16 * 16)
    parts = []
    rem = C
    while rem > 0:
        take = min(rem, maxW2)
        rem -= take
        parts.append(take)
    if with_count and parts[-1] + 16 > maxW2:
        parts[-1] -= 16
        parts.append(16)
        if parts[0] == 0:
            parts = parts[1:]
    accs = []
    off = 0
    cache = {}
    for pi, Cp in enumerate(parts):
        cnt_p = with_count and pi == len(parts) - 1
        kfn, EPAD, CH = _sc_conv(N, E, Cp, cnt_p)
        key = (EPAD, CH)
        if key not in cache:
            pad = EPAD - E
            cache[key] = (
                jnp.pad(e[0], (0, pad)).reshape(EPAD // CH, CH),
                jnp.pad(e[1], (0, pad), constant_values=N
                        ).reshape(EPAD // CH, CH),
                jnp.pad(pseudo[:, 0], (0, pad)),
                jnp.pad(pseudo[:, 1], (0, pad)))
        src2, dst2, p0, p1 = cache[key]
        g_part = g3_pad[:, :, off:off + Cp].reshape(ic, 3 * Cp)
        xg = _tc_matmul(x_in, g_part)
        accs.append(kfn(src2, dst2, p0, p1, coef, xg))
        off += Cp
    nacc = len(accs)
    BR = 256
    acc_specs = [pl.BlockSpec((NC, BR, a.shape[2]), lambda i: (0, i, 0))
                 for a in accs]
    if with_count:
        out, cnt_out = pl.pallas_call(
            functools.partial(_post_body, nacc=nacc, C=C, oc=oc,
                              enc=True, softmax=softmax),
            grid=(pl.cdiv(N, BR),),
            in_specs=acc_specs + [
                pl.BlockSpec((BR, ic), lambda i: (i, 0)),
                pl.BlockSpec((ic, C), lambda i: (0, 0)),
                pl.BlockSpec((1, C), lambda i: (0, 0))],
            out_specs=[pl.BlockSpec((BR, C), lambda i: (i, 0)),
                       pl.BlockSpec((BR, 16), lambda i: (i, 0))],
            out_shape=[jax.ShapeDtypeStruct((N, C), jnp.float32),
                       jax.ShapeDtypeStruct((N, 16), jnp.float32)],
        )(*accs, x_in, root_pad, bias_pad)
        return out[:, :oc], cnt_out
    out = pl.pallas_call(
        functools.partial(_post_body, nacc=nacc, C=C, oc=oc,
                          enc=False, softmax=softmax),
        grid=(pl.cdiv(N, BR),),
        in_specs=acc_specs + [
            pl.BlockSpec((BR, 16), lambda i: (i, 0)),
            pl.BlockSpec((BR, ic), lambda i: (i, 0)),
            pl.BlockSpec((ic, C), lambda i: (0, 0)),
            pl.BlockSpec((1, C), lambda i: (0, 0))],
        out_specs=pl.BlockSpec((BR, C), lambda i: (i, 0)),
        out_shape=jax.ShapeDtypeStruct((N, C), jnp.float32),
    )(*accs, cnt16, x_in, root_pad, bias_pad)
    return out[:, :oc], None


def _hexup(h, up):
    n_new = up.shape[0]
    C = h.shape[1]
    if n_new % (NW * 64) == 0:
        CH = 64
    elif n_new % (NW * 48) == 0:
        CH = 48
    else:
        CH = 16
    kfn, NPAD = _sc_hexup(n_new, C, CH)
    pad = NPAD - n_new
    u0 = jnp.pad(up[:, 0], (0, pad)).reshape(NPAD // CH, CH)
    u1 = jnp.pad(up[:, 1], (0, pad)).reshape(NPAD // CH, CH)
    out = kfn(u0, u1, h)
    return jnp.concatenate([h, out[:n_new]], axis=0)


def _pool(x):
    return x[:(x.shape[0] + 6) // 4]


def kernel(x, e1, e2, e3, e4, e5, e6,
           pseudo1, pseudo2, pseudo3, pseudo4, pseudo5, pseudo6,
           up2, up3, up4, up5, up6, params):
    x0 = x
    h, cnt6 = _gmm(x0, e6, pseudo6, params["conv1"], 32)
    x1 = _pool(h)
    h, cnt5 = _gmm(x1, e5, pseudo5, params["conv2"], 64)
    x2 = _pool(h)
    h, cnt4 = _gmm(x2, e4, pseudo4, params["conv3"], 128)
    x3 = _pool(h)
    h, cnt3 = _gmm(x3, e3, pseudo3, params["conv4"], 256)
    x4 = _pool(h)
    h, cnt2 = _gmm(x4, e2, pseudo2, params["conv5"], 512)
    x5 = _pool(h)
    h, _ = _gmm(x5, e1, pseudo1, params["conv6"], 512)
    h = _hexup(h, up2)
    h = jnp.concatenate([h, x4], axis=1)
    h, _ = _gmm(h, e2, pseudo2, params["conv7"], 256, cnt16=cnt2)
    h = _hexup(h, up3)
    h = jnp.concatenate([h, x3], axis=1)
    h, _ = _gmm(h, e3, pseudo3, params["conv8"], 128, cnt16=cnt3)
    h = _hexup(h, up4)
    h = jnp.concatenate([h, x2], axis=1)
    h, _ = _gmm(h, e4, pseudo4, params["conv9"], 64, cnt16=cnt4)
    h = _hexup(h, up5)
    h = jnp.concatenate([h, x1], axis=1)
    h, _ = _gmm(h, e5, pseudo5, params["conv10"], 32, cnt16=cnt5)
    h = _hexup(h, up6)
    h = jnp.concatenate([h, x0], axis=1)
    h, _ = _gmm(h, e6, pseudo6, params["conv11"], 37, cnt16=cnt6,
                softmax=True)
    return h


# speedup vs baseline: 1.0583x; 1.0002x over previous
import functools

import jax
import jax.numpy as jnp
from jax import lax
from jax.experimental import pallas as pl
from jax.experimental.pallas import tpu as pltpu
from jax.experimental.pallas import tpu_sc as plsc

NC = 2
NS = 16
NW = NC * NS


def _ceil_to(v, m):
    return -(-v // m) * m


def _mm_body(x_ref, w_ref, o_ref):
    o_ref[...] = jnp.dot(x_ref[...], w_ref[...],
                         preferred_element_type=jnp.float32)


def _tc_matmul(x, w, br=512):
    n, ic = x.shape
    oc = w.shape[1]
    return pl.pallas_call(
        _mm_body,
        grid=(pl.cdiv(n, br),),
        in_specs=[pl.BlockSpec((br, ic), lambda i: (i, 0)),
                  pl.BlockSpec((ic, oc), lambda i: (0, 0))],
        out_specs=pl.BlockSpec((br, oc), lambda i: (i, 0)),
        out_shape=jax.ShapeDtypeStruct((n, oc), jnp.float32),
    )(x, w)


def _post_body(*args, nacc, C, oc, enc, softmax):
    acc_parts = [r[0] + r[1] for r in args[:nacc]]
    acc = (jnp.concatenate(acc_parts, axis=1) if nacc > 1
           else acc_parts[0])
    if enc:
        x_ref, r_ref, b_ref, o_ref, c_ref = args[nacc:]
        cnt16 = acc[:, C:C + 16]
        c_ref[...] = cnt16
        cnt = jnp.maximum(cnt16[:, :1], 1.0)
    else:
        cnt_ref, x_ref, r_ref, b_ref, o_ref = args[nacc:]
        cnt = jnp.maximum(cnt_ref[:, :1], 1.0)
    y = acc[:, :C] / cnt
    y = y + jnp.dot(x_ref[...], r_ref[...],
                    preferred_element_type=jnp.float32) + b_ref[...]
    if softmax:
        col = lax.broadcasted_iota(jnp.int32, y.shape, 1)
        y = jnp.where(col < oc, y, -jnp.inf)
        m = jnp.max(y, axis=1, keepdims=True)
        e = jnp.exp(y - m)
        y = e / jnp.sum(e, axis=1, keepdims=True)
    else:
        y = jnp.maximum(y, 0.0)
    o_ref[...] = y


_SPMEM_WORDS = 2_097_151


def _pick_ch(N, E, W, W2):
    R = _ceil_to(N + 1, NS * 8)
    best = 16
    for ch in (128, 64, 32, 16):
        epad = _ceil_to(E, NW * ch * 2)
        epw = epad // NW
        pertile = 4 * epw + 2 * ch * (W + W2) + 3 * (ch + 16) + 1500
        if pertile * 4 > 420_000:
            continue
        if R * W2 + 16 * pertile + 60_000 > _SPMEM_WORDS:
            continue
        return ch
    return best


def _sc_conv(N, E, C, with_count):
    W = 3 * C
    W2 = C + 16 if with_count else C
    CH = _pick_ch(N, E, W, W2)
    EPAD = _ceil_to(E, NW * CH * 2)
    EPW = EPAD // NW
    NCHUNK = EPW // CH
    R = _ceil_to(N + 1, NS * 8)
    RPS = R // NS
    mesh = plsc.VectorSubcoreMesh(core_axis_name="c", subcore_axis_name="s",
                                  num_cores=NC, num_subcores=NS)

    @functools.partial(
        pl.kernel,
        out_type=jax.ShapeDtypeStruct((NC, R, W2), jnp.float32),
        mesh=mesh,
        scratch_types=[
            pltpu.VMEM((NCHUNK, CH), jnp.int32),
            pltpu.VMEM((NCHUNK, CH), jnp.int32),
            pltpu.VMEM((EPW,), jnp.float32),
            pltpu.VMEM((EPW,), jnp.float32),
            pltpu.VMEM((CH, W), jnp.float32),
            pltpu.VMEM((CH, W), jnp.float32),
            pltpu.VMEM((CH, W2), jnp.float32),
            pltpu.VMEM((CH, W2), jnp.float32),
            pltpu.VMEM((3, CH + 16), jnp.float32),
            pltpu.VMEM((12, 16), jnp.float32),
            pltpu.VMEM_SHARED((R, W2), jnp.float32),
            pltpu.SemaphoreType.DMA,
            pltpu.SemaphoreType.DMA,
        ],
        compiler_params=pltpu.CompilerParams(use_tc_tiling_on_sc=False,
                                            needs_layout_passes=False),
    )
    def k(src2_h, dst2_h, p0_h, p1_h, coef_h, xg_h, out_h,
          src_a, dst_a, p0_a, p1_a, rows0, rows1, msg0, msg1, gbuf,
          coef_v, acc_sh, sem0, sem1):
        cid = lax.axis_index("c")
        sid = lax.axis_index("s")
        wid = sid * NC + cid
        rows = (rows0, rows1)
        msgs = (msg0, msg1)
        sems = (sem0, sem1)
        zero16 = jnp.zeros((16,), jnp.float32)

        def zrow(e, carry):
            for cb in range(W2 // 16):
                msg0[e, pl.ds(cb * 16, 16)] = zero16
            return carry

        lax.fori_loop(0, CH, zrow, 0)

        r0 = sid * RPS
        nfull, rem = divmod(RPS, CH)
        for j in range(nfull):
            pltpu.sync_copy(msg0, acc_sh.at[pl.ds(r0 + j * CH, CH)])
        if rem:
            pltpu.sync_copy(msg0.at[pl.ds(0, rem)],
                            acc_sh.at[pl.ds(r0 + nfull * CH, rem)])
        if with_count:
            one16 = jnp.ones((16,), jnp.float32)

            def orow(e, carry):
                msg0[e, pl.ds(C, 16)] = one16
                msg1[e, pl.ds(C, 16)] = one16
                return carry

            lax.fori_loop(0, CH, orow, 0)
        pltpu.sync_copy(coef_h, coef_v)
        pltpu.sync_copy(src2_h.at[pl.ds(wid * NCHUNK, NCHUNK)], src_a)
        pltpu.sync_copy(dst2_h.at[pl.ds(wid * NCHUNK, NCHUNK)], dst_a)
        pltpu.sync_copy(p0_h.at[pl.ds(wid * EPW, EPW)], p0_a)
        pltpu.sync_copy(p1_h.at[pl.ds(wid * EPW, EPW)], p1_a)
        plsc.subcore_barrier()

        pltpu.async_copy(xg_h.at[src_a.at[0]], rows0, sem0)
        pltpu.async_copy(xg_h.at[src_a.at[1]], rows1, sem1)

        def pair(i, carry):
            for b in (0, 1):
                ci = 2 * i + b
                pltpu.make_async_copy(xg_h.at[src_a.at[b]], rows[b],
                                      sems[b]).wait()
                off = ci * CH
                rv = rows[b]
                mv = msgs[b]
                for j in range(CH // 16):
                    p0 = p0_a[pl.ds(off + j * 16, 16)]
                    p1 = p1_a[pl.ds(off + j * 16, 16)]
                    for kk in range(3):
                        d0 = p0 - coef_v[kk]
                        d1 = p1 - coef_v[3 + kk]
                        gbuf[kk, pl.ds(j * 16, 16)] = jnp.exp(
                            coef_v[6 + kk] * d0 * d0
                            + coef_v[9 + kk] * d1 * d1)

                def erow(e, c2):
                    g0 = gbuf[0, pl.ds(e, 16)][0]
                    g1 = gbuf[1, pl.ds(e, 16)][0]
                    g2 = gbuf[2, pl.ds(e, 16)][0]
                    for cb in range(C // 16):
                        v = (g0 * rv[e, pl.ds(cb * 16, 16)]
                             + g1 * rv[e, pl.ds(C + cb * 16, 16)]
                             + g2 * rv[e, pl.ds(2 * C + cb * 16, 16)])
                        mv[e, pl.ds(cb * 16, 16)] = v
                    return c2

                lax.fori_loop(0, CH, erow, 0,
                              unroll=(8 if C <= 32 else
                                      4 if C <= 128 else 1))
                pltpu.sync_copy(mv, acc_sh.at[dst_a.at[ci]], add=True)
                if NCHUNK > 2:
                    @pl.when(ci + 2 < NCHUNK)
                    def _():
                        pltpu.async_copy(xg_h.at[src_a.at[ci + 2]],
                                         rows[b], sems[b])
            return carry

        lax.fori_loop(0, NCHUNK // 2, pair, 0)
        plsc.subcore_barrier()
        pltpu.sync_copy(acc_sh.at[pl.ds(r0, RPS)],
                        out_h.at[cid, pl.ds(r0, RPS)])

    return k, EPAD, CH


def _sc_hexup(n_new, C, CH):
    NPAD = _ceil_to(n_new, NW * CH)
    NPW = NPAD // NW
    NCHUNK = NPW // CH
    mesh = plsc.VectorSubcoreMesh(core_axis_name="c", subcore_axis_name="s",
                                  num_cores=NC, num_subcores=NS)

    @functools.partial(
        pl.kernel,
        out_type=jax.ShapeDtypeStruct((NPAD, C), jnp.float32),
        mesh=mesh,
        scratch_types=[
            pltpu.VMEM((NCHUNK, CH), jnp.int32),
            pltpu.VMEM((NCHUNK, CH), jnp.int32),
            pltpu.VMEM((CH, C), jnp.float32),
            pltpu.VMEM((CH, C), jnp.float32),
            pltpu.SemaphoreType.DMA,
        ],
        compiler_params=pltpu.CompilerParams(use_tc_tiling_on_sc=False,
                                            needs_layout_passes=False),
    )
    def k(u02_h, u12_h, h_h, out_h, u0_a, u1_a, r0_v, r1_v, sem):
        cid = lax.axis_index("c")
        sid = lax.axis_index("s")
        wid = sid * NC + cid
        base_w = wid * NPW
        pltpu.sync_copy(u02_h.at[pl.ds(wid * NCHUNK, NCHUNK)], u0_a)
        pltpu.sync_copy(u12_h.at[pl.ds(wid * NCHUNK, NCHUNK)], u1_a)

        def chunk(ci, carry):
            pltpu.async_copy(h_h.at[u0_a.at[ci]], r0_v, sem)
            pltpu.async_copy(h_h.at[u1_a.at[ci]], r1_v, sem)
            pltpu.make_async_copy(h_h.at[u0_a.at[ci]], r0_v, sem).wait()
            pltpu.make_async_copy(h_h.at[u1_a.at[ci]], r1_v, sem).wait()

            def erow(e, c2):
                for cb in range(C // 16):
                    s = pl.ds(cb * 16, 16)
                    r0_v[e, s] = (r0_v[e, s] + r1_v[e, s]) * 0.5
                return c2

            lax.fori_loop(0, CH, erow, 0)
            pltpu.sync_copy(r0_v, out_h.at[pl.ds(base_w + ci * CH, CH)])
            return carry

        lax.fori_loop(0, NCHUNK, chunk, 0)

    return k, NPAD


def _gmm_prep(p, oc):
    ic = p["g"].shape[0]
    C = _ceil_to(oc, 16)
    g3 = p["g"].reshape(ic, 3, oc)
    g_pad = jnp.zeros((ic, 3, C), jnp.float32).at[:, :, :oc].set(g3)
    root_pad = jnp.zeros((ic, C), jnp.float32).at[:, :oc].set(p["root"])
    bias_pad = jnp.zeros((1, C), jnp.float32).at[0, :oc].set(p["bias"])
    a = -0.5 / (1e-14 + p["sigma"] ** 2)
    coef = jnp.concatenate([p["mu"][:, 0], p["mu"][:, 1], a[:, 0], a[:, 1]])
    coef = jnp.broadcast_to(coef[:, None], (12, 16)).astype(jnp.float32)
    return C, g_pad, root_pad, bias_pad, coef


_ACC_BUDGET_WORDS = 1_350_000


def _gmm(x_in, e, pseudo, p, oc, cnt16=None, softmax=False):
    N, ic = x_in.shape
    E = e.shape[1]
    C, g3_pad, root_pad, bias_pad, coef = _gmm_prep(p, oc)
    with_count = cnt16 is None
    R = _ceil_to(N + 1, NS * 8)
    maxW2 = max(16, (_ACC_BUDGET_WORDS // R) // 16 * 16)
    parts = []
    rem = C
    while rem > 0:
        take = min(rem, maxW2)
        rem -= take
        parts.append(take)
    if with_count and parts[-1] + 16 > maxW2:
        parts[-1] -= 16
        parts.append(16)
        if parts[0] == 0:
            parts = parts[1:]
    accs = []
    off = 0
    cache = {}
    for pi, Cp in enumerate(parts):
        cnt_p = with_count and pi == len(parts) - 1
        kfn, EPAD, CH = _sc_conv(N, E, Cp, cnt_p)
        key = (EPAD, CH)
        if key not in cache:
            pad = EPAD - E
            cache[key] = (
                jnp.pad(e[0], (0, pad)).reshape(EPAD // CH, CH),
                jnp.pad(e[1], (0, pad), constant_values=N
                        ).reshape(EPAD // CH, CH),
                jnp.pad(pseudo[:, 0], (0, pad)),
                jnp.pad(pseudo[:, 1], (0, pad)))
        src2, dst2, p0, p1 = cache[key]
        g_part = g3_pad[:, :, off:off + Cp].reshape(ic, 3 * Cp)
        xg = _tc_matmul(x_in, g_part)
        accs.append(kfn(src2, dst2, p0, p1, coef, xg))
        off += Cp
    nacc = len(accs)
    BR = 256
    acc_specs = [pl.BlockSpec((NC, BR, a.shape[2]), lambda i: (0, i, 0))
                 for a in accs]
    if with_count:
        out, cnt_out = pl.pallas_call(
            functools.partial(_post_body, nacc=nacc, C=C, oc=oc,
                              enc=True, softmax=softmax),
            grid=(pl.cdiv(N, BR),),
            in_specs=acc_specs + [
                pl.BlockSpec((BR, ic), lambda i: (i, 0)),
                pl.BlockSpec((ic, C), lambda i: (0, 0)),
                pl.BlockSpec((1, C), lambda i: (0, 0))],
            out_specs=[pl.BlockSpec((BR, C), lambda i: (i, 0)),
                       pl.BlockSpec((BR, 16), lambda i: (i, 0))],
            out_shape=[jax.ShapeDtypeStruct((N, C), jnp.float32),
                       jax.ShapeDtypeStruct((N, 16), jnp.float32)],
        )(*accs, x_in, root_pad, bias_pad)
        return out[:, :oc], cnt_out
    out = pl.pallas_call(
        functools.partial(_post_body, nacc=nacc, C=C, oc=oc,
                          enc=False, softmax=softmax),
        grid=(pl.cdiv(N, BR),),
        in_specs=acc_specs + [
            pl.BlockSpec((BR, 16), lambda i: (i, 0)),
            pl.BlockSpec((BR, ic), lambda i: (i, 0)),
            pl.BlockSpec((ic, C), lambda i: (0, 0)),
            pl.BlockSpec((1, C), lambda i: (0, 0))],
        out_specs=pl.BlockSpec((BR, C), lambda i: (i, 0)),
        out_shape=jax.ShapeDtypeStruct((N, C), jnp.float32),
    )(*accs, cnt16, x_in, root_pad, bias_pad)
    return out[:, :oc], None


def _hexup(h, up):
    n_new = up.shape[0]
    C = h.shape[1]
    if n_new % (NW * 64) == 0:
        CH = 64
    elif n_new % (NW * 48) == 0:
        CH = 48
    else:
        CH = 16
    kfn, NPAD = _sc_hexup(n_new, C, CH)
    pad = NPAD - n_new
    u0 = jnp.pad(up[:, 0], (0, pad)).reshape(NPAD // CH, CH)
    u1 = jnp.pad(up[:, 1], (0, pad)).reshape(NPAD // CH, CH)
    out = kfn(u0, u1, h)
    return jnp.concatenate([h, out[:n_new]], axis=0)


def _pool(x):
    return x[:(x.shape[0] + 6) // 4]


def kernel(x, e1, e2, e3, e4, e5, e6,
           pseudo1, pseudo2, pseudo3, pseudo4, pseudo5, pseudo6,
           up2, up3, up4, up5, up6, params):
    x0 = x
    h, cnt6 = _gmm(x0, e6, pseudo6, params["conv1"], 32)
    x1 = _pool(h)
    h, cnt5 = _gmm(x1, e5, pseudo5, params["conv2"], 64)
    x2 = _pool(h)
    h, cnt4 = _gmm(x2, e4, pseudo4, params["conv3"], 128)
    x3 = _pool(h)
    h, cnt3 = _gmm(x3, e3, pseudo3, params["conv4"], 256)
    x4 = _pool(h)
    h, cnt2 = _gmm(x4, e2, pseudo2, params["conv5"], 512)
    x5 = _pool(h)
    h, _ = _gmm(x5, e1, pseudo1, params["conv6"], 512)
    h = _hexup(h, up2)
    h = jnp.concatenate([h, x4], axis=1)
    h, _ = _gmm(h, e2, pseudo2, params["conv7"], 256, cnt16=cnt2)
    h = _hexup(h, up3)
    h = jnp.concatenate([h, x3], axis=1)
    h, _ = _gmm(h, e3, pseudo3, params["conv8"], 128, cnt16=cnt3)
    h = _hexup(h, up4)
    h = jnp.concatenate([h, x2], axis=1)
    h, _ = _gmm(h, e4, pseudo4, params["conv9"], 64, cnt16=cnt4)
    h = _hexup(h, up5)
    h = jnp.concatenate([h, x1], axis=1)
    h, _ = _gmm(h, e5, pseudo5, params["conv10"], 32, cnt16=cnt5)
    h = _hexup(h, up6)
    h = jnp.concatenate([h, x0], axis=1)
    h, _ = _gmm(h, e6, pseudo6, params["conv11"], 37, cnt16=cnt6,
                softmax=True)
    return h
